# baseline (speedup 1.0000x reference)
import functools

import jax
import jax.numpy as jnp
from jax import lax
from jax.experimental import pallas as pl
from jax.experimental.pallas import tpu as pltpu
from jax.experimental.pallas import tpu_sc as plsc

NUM_WORKERS = 32
LANES = 16
CHUNK = 128


def _make_kernel(batch, dim):
  b_per_w = batch // NUM_WORKERS
  n_chunks = b_per_w // CHUNK
  mesh = plsc.VectorSubcoreMesh(core_axis_name="c", subcore_axis_name="s")

  @functools.partial(
      pl.kernel,
      out_type=jax.ShapeDtypeStruct((batch,), jnp.float32),
      mesh=mesh,
      compiler_params=pltpu.CompilerParams(
          needs_layout_passes=False, use_tc_tiling_on_sc=False),
      scratch_types=[
          pltpu.VMEM((n_chunks, CHUNK), jnp.int32),
          pltpu.VMEM((n_chunks, CHUNK), jnp.int32),
          pltpu.VMEM((b_per_w,), jnp.int32),
          pltpu.VMEM((b_per_w,), jnp.int32),
          pltpu.VMEM((2, CHUNK, 2 * dim), jnp.float32),
          pltpu.VMEM((2, CHUNK, 2 * dim), jnp.float32),
          pltpu.VMEM((b_per_w,), jnp.float32),
          pltpu.SemaphoreType.DMA,
          pltpu.SemaphoreType.DMA,
          pltpu.SemaphoreType.DMA,
          pltpu.SemaphoreType.DMA,
      ],
  )
  def k(ug_hbm, ig_hbm, up_hbm, ip_hbm, ue_hbm, ie_hbm, out_hbm,
        ugidx, igidx, upar, ipar, urows, irows, outv, us0, us1, is0, is1):
    usems = (us0, us1)
    isems = (is0, is1)
    wid = lax.axis_index("s") * 2 + lax.axis_index("c")
    base = wid * b_per_w

    pltpu.sync_copy(ug_hbm.at[wid], ugidx)
    pltpu.sync_copy(ig_hbm.at[wid], igidx)
    pltpu.sync_copy(up_hbm.at[wid], upar)
    pltpu.sync_copy(ip_hbm.at[wid], ipar)

    def fire(c):
      buf = c % 2
      du = pltpu.async_copy(ue_hbm.at[ugidx.at[c]], urows.at[buf], usems[buf])
      di = pltpu.async_copy(ie_hbm.at[igidx.at[c]], irows.at[buf], isems[buf])
      return du, di

    r_iota = lax.iota(jnp.int32, LANES)
    inflight = fire(0)

    for c in range(n_chunks):
      du, di = inflight
      if c + 1 < n_chunks:
        nxt = fire(c + 1)
      du.wait()
      di.wait()
      buf = c % 2
      ubuf = urows.at[buf]
      ibuf = irows.at[buf]

      def block_body(bi, _, c=c, ubuf=ubuf, ibuf=ibuf):
        rows = r_iota + bi * LANES
        ub = upar[pl.ds(c * CHUNK + bi * LANES, LANES)]
        ib = ipar[pl.ds(c * CHUNK + bi * LANES, LANES)]

        def col_body(j, acc):
          jv = jnp.full((LANES,), 0, jnp.int32) + j
          uvec = plsc.load_gather(ubuf, [rows, ub + jv])
          ivec = plsc.load_gather(ibuf, [rows, ib + jv])
          return acc + uvec * ivec

        acc = lax.fori_loop(0, dim, col_body,
                            jnp.zeros((LANES,), jnp.float32))
        outv[pl.ds(c * CHUNK + bi * LANES, LANES)] = acc
        return 0

      lax.fori_loop(0, CHUNK // LANES, block_body, 0)
      if c + 1 < n_chunks:
        inflight = nxt

    pltpu.sync_copy(outv, out_hbm.at[pl.ds(base, b_per_w)])

  return k


def kernel(user, item, user_emb, item_emb):
  batch = user.shape[0]
  n_rows, dim = user_emb.shape
  b_per_w = batch // NUM_WORKERS
  n_chunks = b_per_w // CHUNK
  ue = user_emb.reshape(n_rows // 2, 2 * dim)
  ie = item_emb.reshape(n_rows // 2, 2 * dim)
  u = user.astype(jnp.int32)
  i = item.astype(jnp.int32)
  ug = (u >> 1).reshape(NUM_WORKERS, n_chunks, CHUNK)
  ig = (i >> 1).reshape(NUM_WORKERS, n_chunks, CHUNK)
  up = ((u & 1) * dim).reshape(NUM_WORKERS, b_per_w)
  ip = ((i & 1) * dim).reshape(NUM_WORKERS, b_per_w)
  k = _make_kernel(batch, dim)
  return k(ug, ig, up, ip, ue, ie)


# speedup vs baseline: 1.0666x; 1.0666x over previous
import functools

import jax
import jax.numpy as jnp
from jax import lax
from jax.experimental import pallas as pl
from jax.experimental.pallas import tpu as pltpu
from jax.experimental.pallas import tpu_sc as plsc

NUM_WORKERS = 32
LANES = 16
CHUNK = 128


def _make_kernel(batch, dim):
  b_per_w = batch // NUM_WORKERS
  n_chunks = b_per_w // CHUNK
  mesh = plsc.VectorSubcoreMesh(core_axis_name="c", subcore_axis_name="s")

  @functools.partial(
      pl.kernel,
      out_type=jax.ShapeDtypeStruct((batch,), jnp.float32),
      mesh=mesh,
      compiler_params=pltpu.CompilerParams(
          needs_layout_passes=False, use_tc_tiling_on_sc=False),
      scratch_types=[
          pltpu.VMEM((n_chunks, CHUNK), jnp.int32),
          pltpu.VMEM((n_chunks, CHUNK), jnp.int32),
          pltpu.VMEM((2, CHUNK, 2 * dim), jnp.float32),
          pltpu.VMEM((2, CHUNK, 2 * dim), jnp.float32),
          pltpu.VMEM((b_per_w,), jnp.float32),
          pltpu.SemaphoreType.DMA,
          pltpu.SemaphoreType.DMA,
          pltpu.SemaphoreType.DMA,
          pltpu.SemaphoreType.DMA,
      ],
  )
  def k(ug_hbm, ig_hbm, ue_hbm, ie_hbm, out_hbm,
        ugidx, igidx, urows, irows, outv, us0, us1, is0, is1):
    usems = (us0, us1)
    isems = (is0, is1)
    wid = lax.axis_index("s") * 2 + lax.axis_index("c")
    base = wid * b_per_w

    pltpu.sync_copy(ug_hbm.at[wid], ugidx)
    pltpu.sync_copy(ig_hbm.at[wid], igidx)

    def fire(c):
      buf = c % 2
      du = pltpu.async_copy(ue_hbm.at[ugidx.at[c]], urows.at[buf], usems[buf])
      di = pltpu.async_copy(ie_hbm.at[igidx.at[c]], irows.at[buf], isems[buf])
      return du, di

    r_iota = lax.iota(jnp.int32, LANES)
    inflight = fire(0)

    for c in range(n_chunks):
      du, di = inflight
      if c + 1 < n_chunks:
        nxt = fire(c + 1)
      du.wait()
      di.wait()
      buf = c % 2
      ubuf = urows.at[buf]
      ibuf = irows.at[buf]

      def block_body(bi, _, c=c, ubuf=ubuf, ibuf=ibuf):
        rows = r_iota + bi * LANES

        def col_body(j, acc):
          jv = jnp.full((LANES,), 0, jnp.int32) + j
          uvec = plsc.load_gather(ubuf, [rows, jv])
          ivec = plsc.load_gather(ibuf, [rows, jv])
          return acc + uvec * ivec

        acc = lax.fori_loop(0, dim, col_body,
                            jnp.zeros((LANES,), jnp.float32))
        outv[pl.ds(c * CHUNK + bi * LANES, LANES)] = acc
        return 0

      lax.fori_loop(0, CHUNK // LANES, block_body, 0)
      if c + 1 < n_chunks:
        inflight = nxt

    pltpu.sync_copy(outv, out_hbm.at[pl.ds(base, b_per_w)])

  return k


def kernel(user, item, user_emb, item_emb):
  batch = user.shape[0]
  n_rows, dim = user_emb.shape
  b_per_w = batch // NUM_WORKERS
  n_chunks = b_per_w // CHUNK
  ue = jnp.pad(user_emb, ((0, 0), (0, dim)))
  ie = jnp.pad(item_emb, ((0, 0), (0, dim)))
  u = user.astype(jnp.int32)
  i = item.astype(jnp.int32)
  ug = u.reshape(NUM_WORKERS, n_chunks, CHUNK)
  ig = i.reshape(NUM_WORKERS, n_chunks, CHUNK)
  k = _make_kernel(batch, dim)
  return k(ug, ig, ue, ie)


# speedup vs baseline: 2.0559x; 1.9276x over previous
import functools

import jax
import jax.numpy as jnp
from jax import lax
from jax.experimental import pallas as pl
from jax.experimental.pallas import tpu as pltpu
from jax.experimental.pallas import tpu_sc as plsc

NUM_WORKERS = 32
LANES = 16
ROWS_PER_CHUNK = 512
SEG_CHUNKS = 16
CAND_CAP = 1024

_GATHER_DNUMS = lax.GatherDimensionNumbers(
    offset_dims=(), collapsed_slice_dims=(0,), start_index_map=(0,))


def _splat_lane(vec, lane_splat):
  return lax.gather(
      vec, lane_splat[:, None], _GATHER_DNUMS, (1,),
      mode=lax.GatherScatterMode.PROMISE_IN_BOUNDS)


def _scalar(splat16):
  return jnp.max(splat16, axis=0)


def _make_extract(n_rows, dim, batch):
  n_chunks_full = n_rows // ROWS_PER_CHUNK
  tail = n_rows - n_chunks_full * ROWS_PER_CHUNK
  per_tec = n_chunks_full // NUM_WORKERS
  leftover = n_chunks_full - per_tec * NUM_WORKERS
  n_extra = leftover + (1 if tail else 0)
  assert n_extra <= 2
  mesh = plsc.VectorSubcoreMesh(core_axis_name="c", subcore_axis_name="s")

  seg_sizes = []
  left = per_tec
  while left > 0:
    seg_sizes.append(min(SEG_CHUNKS, left))
    left -= seg_sizes[-1]

  @functools.partial(
      pl.kernel,
      out_type=(jax.ShapeDtypeStruct((batch, 2 * dim), jnp.float32),
                jax.ShapeDtypeStruct((batch, 2 * dim), jnp.float32)),
      mesh=mesh,
      compiler_params=pltpu.CompilerParams(
          needs_layout_passes=False, use_tc_tiling_on_sc=True),
      scratch_types=[
          pltpu.VMEM((batch,), jnp.int32),
          pltpu.VMEM((CAND_CAP + LANES,), jnp.int32),
          pltpu.VMEM((CAND_CAP + LANES,), jnp.int32),
          pltpu.VMEM((2, dim, ROWS_PER_CHUNK), jnp.float32),
          pltpu.VMEM((LANES, 2 * dim), jnp.float32),
          pltpu.VMEM((LANES,), jnp.int32),
          pltpu.SemaphoreType.DMA,
          pltpu.SemaphoreType.DMA,
          pltpu.SemaphoreType.DMA,
      ],
  )
  def k(ut_hbm, it_hbm, tu_hbm, ti_hbm, uidx_hbm, iidx_hbm, eu_hbm, ei_hbm,
        idxv, cr, cb, bufs, stage, stageb, s0, s1, ssc):
    wid = lax.axis_index("s") * 2 + lax.axis_index("c")
    sems = (s0, s1)
    lane_iota = lax.iota(jnp.int32, LANES)
    fdim = dim // LANES

    def run_table(tab_hbm, tail_hbm, idx_hbm, ext_hbm):
      pltpu.sync_copy(idx_hbm, idxv)

      def fire(chunk_id, par):
        off = pl.multiple_of(chunk_id * ROWS_PER_CHUNK, ROWS_PER_CHUNK)

        def go(b):
          def f(_):
            pltpu.async_copy(
                tab_hbm.at[:, pl.ds(off, ROWS_PER_CHUNK)], bufs.at[b],
                sems[b])
            return 0
          return f
        lax.cond(par == 0, go(0), go(1), 0)

      def wait_chunk(par):
        def go(b):
          def f(_):
            pltpu.make_async_copy(
                tab_hbm.at[:, pl.ds(0, ROWS_PER_CHUNK)], bufs.at[b],
                sems[b]).wait()
            return 0
          return f
        lax.cond(par == 0, go(0), go(1), 0)

      def flush(state):
        nslot, bacc = state
        stageb[...] = jnp.where(lane_iota < nslot, bacc, -1)
        pltpu.async_copy(
            stage, ext_hbm.at[plsc.Indices(stageb, ignored_value=-1)],
            ssc).wait()
        return (jnp.int32(0), jnp.full((LANES,), -1, jnp.int32))

      def extract_matches(match_lo, match_hi, col_base, buf_par, n_cand,
                          state):
        par_v = jnp.full((LANES,), 0, jnp.int32) + buf_par

        def cand_iter(ci, state):
          rv = cr[pl.ds(ci * LANES, LANES)]
          bv = cb[pl.ds(ci * LANES, LANES)]
          live = lane_iota < (n_cand - ci * LANES)
          m = jnp.logical_and((rv >= match_lo) & (rv < match_hi), live)

          def has_more(carry):
            m, _ = carry
            return jnp.any(m)

          def pull_one(carry):
            m, state = carry
            nslot, bacc = state
            l = plsc.all_reduce_ffs(m)
            r_s = _splat_lane(rv, l)
            b_s = _splat_lane(bv, l)
            col = r_s - col_base
            nslot_v = jnp.full((LANES,), 0, jnp.int32) + nslot
            for q in range(fdim):
              feat = lane_iota + q * LANES
              v = plsc.load_gather(bufs, [par_v, feat, col])
              plsc.store_scatter(stage, [nslot_v, feat], v)
            bacc = jnp.where(lane_iota == nslot, b_s, bacc)
            m = jnp.logical_and(m, lane_iota != l)
            state = (nslot + jnp.int32(1), bacc)
            state = lax.cond(state[0] == LANES, flush, lambda s: s, state)
            return m, state

          _, state = lax.while_loop(has_more, pull_one, (m, state))
          return state

        n_iter = lax.div(n_cand + jnp.int32(LANES - 1), jnp.int32(LANES))
        return lax.fori_loop(0, n_iter, cand_iter, state)

      def compress_segment(lo, hi, start):
        def body(i, carry):
          cnt, resume, full = carry
          base = i * LANES
          rv = idxv[pl.ds(base, LANES)]
          m = (rv >= lo) & (rv < hi)
          m = jnp.logical_and(m, base >= start)
          m = jnp.logical_and(m, jnp.logical_not(full))
          npos = _scalar(plsc.all_reduce_population_count(m))
          can = jnp.logical_and(
              jnp.logical_not(full), cnt + npos <= CAND_CAP)
          can = jnp.logical_and(can, base >= start)

          def do_store(cnt):
            plsc.store_compressed(cr.at[pl.ds(cnt, LANES)], rv, mask=m)
            plsc.store_compressed(
                cb.at[pl.ds(cnt, LANES)], lane_iota + base, mask=m)
            return cnt + npos

          cnt = lax.cond(can, do_store, lambda c: c, cnt)
          newfull = jnp.logical_and(base >= start,
                                    jnp.logical_not(can))
          newfull = jnp.logical_and(newfull, jnp.logical_not(full))
          resume = lax.select(newfull, base, resume)
          full = jnp.logical_or(full, newfull)
          return cnt, resume, full

        cnt, resume, _ = lax.fori_loop(
            0, batch // LANES, body,
            (jnp.int32(0), jnp.int32(batch), jnp.bool_(False)))
        return cnt, resume

      state = (jnp.int32(0), jnp.full((LANES,), -1, jnp.int32))
      chunk0 = wid * per_tec
      n_seg = len(seg_sizes)

      def seg_body(s, state):
        seg_len = jnp.minimum(
            jnp.int32(SEG_CHUNKS), jnp.int32(per_tec) - s * SEG_CHUNKS)
        lo_chunk = chunk0 + s * SEG_CHUNKS
        lo = lo_chunk * ROWS_PER_CHUNK
        hi = lo + seg_len * ROWS_PER_CHUNK

        def not_done(carry):
          return carry[0] < batch

        def one_pass(carry):
          start, state = carry
          n_cand, resume = compress_segment(lo, hi, start)
          fire(lo_chunk, 0)

          def jbody(j, state):
            par = lax.rem(j, 2)
            lax.cond(j + 1 < seg_len,
                     lambda _: (fire(lo_chunk + j + 1, 1 - par), 0)[1],
                     lambda _: 0, 0)
            wait_chunk(par)
            clo = (lo_chunk + j) * ROWS_PER_CHUNK
            return extract_matches(clo, clo + ROWS_PER_CHUNK, clo,
                                   par, n_cand, state)

          state = lax.fori_loop(0, seg_len, jbody, state)
          return resume, state

        return lax.while_loop(not_done, one_pass, (jnp.int32(0), state))[1]

      state = lax.fori_loop(0, n_seg, seg_body, state)

      if n_extra:
        def leftover_fn(state):
          is_tail = wid == NUM_WORKERS - 1
          full_off = (n_chunks_full - 1) * ROWS_PER_CHUNK
          lo = lax.select(is_tail, jnp.int32(n_chunks_full * ROWS_PER_CHUNK),
                          jnp.int32(full_off))
          hi = lax.select(is_tail, jnp.int32(n_rows),
                          jnp.int32(n_chunks_full * ROWS_PER_CHUNK))
          col_base = lax.select(is_tail, jnp.int32(n_rows - ROWS_PER_CHUNK),
                                jnp.int32(full_off))

          def not_done(carry):
            return carry[0] < batch

          def one_pass(carry):
            start, state = carry
            n_cand, resume = compress_segment(lo, hi, start)

            def dma_full(_):
              pltpu.async_copy(
                  tab_hbm.at[:, pl.ds(full_off, ROWS_PER_CHUNK)],
                  bufs.at[0], sems[0]).wait()
              return 0

            def dma_tail(_):
              pltpu.async_copy(tail_hbm, bufs.at[0], sems[0]).wait()
              return 0

            lax.cond(is_tail, dma_tail, dma_full, 0)
            state = extract_matches(lo, hi, col_base, jnp.int32(0),
                                    n_cand, state)
            return resume, state

          return lax.while_loop(not_done, one_pass,
                                (jnp.int32(0), state))[1]

        state = lax.cond(wid >= NUM_WORKERS - n_extra, leftover_fn,
                         lambda s: s, state)

      state = lax.cond(state[0] > 0, flush, lambda s: s, state)

    run_table(ut_hbm, tu_hbm, uidx_hbm, eu_hbm)
    run_table(it_hbm, ti_hbm, iidx_hbm, ei_hbm)

  return k


def _make_compute(batch, dim):
  b_per_w = batch // NUM_WORKERS
  half = b_per_w // 2
  mesh = plsc.VectorSubcoreMesh(core_axis_name="c", subcore_axis_name="s")

  @functools.partial(
      pl.kernel,
      out_type=jax.ShapeDtypeStruct((batch,), jnp.float32),
      mesh=mesh,
      compiler_params=pltpu.CompilerParams(
          needs_layout_passes=False, use_tc_tiling_on_sc=True),
      scratch_types=[
          pltpu.VMEM((half, 2 * dim), jnp.float32),
          pltpu.VMEM((half, 2 * dim), jnp.float32),
          pltpu.VMEM((b_per_w,), jnp.float32),
          pltpu.SemaphoreType.DMA,
          pltpu.SemaphoreType.DMA,
      ],
  )
  def k(eu_hbm, ei_hbm, out_hbm, ubuf, ibuf, outv, su, si):
    wid = lax.axis_index("s") * 2 + lax.axis_index("c")
    base = wid * b_per_w
    r_iota = lax.iota(jnp.int32, LANES)

    for h in range(2):
      off = base + h * half
      du = pltpu.async_copy(eu_hbm.at[pl.ds(off, half)], ubuf, su)
      di = pltpu.async_copy(ei_hbm.at[pl.ds(off, half)], ibuf, si)
      du.wait()
      di.wait()

      def block_body(bi, _, h=h):
        rows = r_iota + bi * LANES

        def col_body(j, acc):
          jv = jnp.full((LANES,), 0, jnp.int32) + j
          uvec = plsc.load_gather(ubuf, [rows, jv])
          ivec = plsc.load_gather(ibuf, [rows, jv])
          return acc + uvec * ivec

        acc = lax.fori_loop(0, dim, col_body,
                            jnp.zeros((LANES,), jnp.float32))
        outv[pl.ds(h * half + bi * LANES, LANES)] = acc
        return 0

      lax.fori_loop(0, half // LANES, block_body, 0)

    pltpu.sync_copy(outv, out_hbm.at[pl.ds(base, b_per_w)])

  return k


def kernel(user, item, user_emb, item_emb):
  batch = user.shape[0]
  n_rows, dim = user_emb.shape
  ut = jnp.swapaxes(user_emb, 0, 1)
  it = jnp.swapaxes(item_emb, 0, 1)
  u = user.astype(jnp.int32)
  i = item.astype(jnp.int32)
  tail = n_rows % ROWS_PER_CHUNK
  def tail_stage(tab):
    t = lax.slice(tab, (n_rows - tail, 0), (n_rows, dim))
    t = jnp.swapaxes(t, 0, 1)
    z = jnp.zeros((dim, ROWS_PER_CHUNK), jnp.float32)
    return lax.dynamic_update_slice(z, t, (0, ROWS_PER_CHUNK - tail))
  tu = tail_stage(user_emb)
  ti = tail_stage(item_emb)
  ka = _make_extract(n_rows, dim, batch)
  eu, ei = ka(ut, it, tu, ti, u, i)
  kb = _make_compute(batch, dim)
  return kb(eu, ei)


# speedup vs baseline: 2.0876x; 1.0154x over previous
import functools

import jax
import jax.numpy as jnp
from jax import lax
from jax.experimental import pallas as pl
from jax.experimental.pallas import tpu as pltpu
from jax.experimental.pallas import tpu_sc as plsc

NUM_WORKERS = 32
LANES = 16
ROWS_PER_CHUNK = 512
SEG_CHUNKS = 16
CAND_CAP = 1024

_GATHER_DNUMS = lax.GatherDimensionNumbers(
    offset_dims=(), collapsed_slice_dims=(0,), start_index_map=(0,))


def _splat_lane(vec, lane_splat):
  return lax.gather(
      vec, lane_splat[:, None], _GATHER_DNUMS, (1,),
      mode=lax.GatherScatterMode.PROMISE_IN_BOUNDS)


def _lane0(x):
  return lax.squeeze(lax.slice(x, (0,), (1,)), (0,))


def _make_extract(n_rows, dim, batch):
  n_chunks_full = n_rows // ROWS_PER_CHUNK
  tail = n_rows - n_chunks_full * ROWS_PER_CHUNK
  per_tec = n_chunks_full // NUM_WORKERS
  leftover = n_chunks_full - per_tec * NUM_WORKERS
  n_extra = leftover + (1 if tail else 0)
  assert n_extra <= 2
  mesh = plsc.VectorSubcoreMesh(core_axis_name="c", subcore_axis_name="s")

  seg_sizes = []
  left = per_tec
  while left > 0:
    seg_sizes.append(min(SEG_CHUNKS, left))
    left -= seg_sizes[-1]

  @functools.partial(
      pl.kernel,
      out_type=(jax.ShapeDtypeStruct((batch, 2 * dim), jnp.float32),
                jax.ShapeDtypeStruct((batch, 2 * dim), jnp.float32)),
      mesh=mesh,
      compiler_params=pltpu.CompilerParams(
          needs_layout_passes=False, use_tc_tiling_on_sc=True),
      scratch_types=[
          pltpu.VMEM((batch,), jnp.int32),
          pltpu.VMEM((CAND_CAP + LANES,), jnp.int32),
          pltpu.VMEM((CAND_CAP + LANES,), jnp.int32),
          pltpu.VMEM((2, dim, ROWS_PER_CHUNK), jnp.float32),
          pltpu.VMEM((LANES, 2 * dim), jnp.float32),
          pltpu.VMEM((LANES,), jnp.int32),
          pltpu.SemaphoreType.DMA,
          pltpu.SemaphoreType.DMA,
          pltpu.SemaphoreType.DMA,
      ],
  )
  def k(ut_hbm, it_hbm, tu_hbm, ti_hbm, uidx_hbm, iidx_hbm, eu_hbm, ei_hbm,
        idxv, cr, cb, bufs, stage, stageb, s0, s1, ssc):
    wid = lax.axis_index("s") * 2 + lax.axis_index("c")
    sems = (s0, s1)
    lane_iota = lax.iota(jnp.int32, LANES)
    fdim = dim // LANES

    def run_table(tab_hbm, tail_hbm, idx_hbm, ext_hbm):
      pltpu.sync_copy(idx_hbm, idxv)

      def fire(chunk_id, par):
        off = pl.multiple_of(chunk_id * ROWS_PER_CHUNK, ROWS_PER_CHUNK)

        def go(b):
          def f(_):
            pltpu.async_copy(
                tab_hbm.at[:, pl.ds(off, ROWS_PER_CHUNK)], bufs.at[b],
                sems[b])
            return 0
          return f
        lax.cond(par == 0, go(0), go(1), 0)

      def wait_chunk(par):
        def go(b):
          def f(_):
            pltpu.make_async_copy(
                tab_hbm.at[:, pl.ds(0, ROWS_PER_CHUNK)], bufs.at[b],
                sems[b]).wait()
            return 0
          return f
        lax.cond(par == 0, go(0), go(1), 0)

      def flush(nslot):
        sb = stageb[...]
        stageb[...] = jnp.where(lane_iota < nslot, sb, -1)
        pltpu.async_copy(
            stage, ext_hbm.at[plsc.Indices(stageb, ignored_value=-1)],
            ssc).wait()
        return jnp.int32(0)

      def extract_matches(match_lo, match_hi, col_base, buf_par, n_cand,
                          state):
        par_v = jnp.full((LANES,), 0, jnp.int32) + buf_par

        def cand_iter(ci, nslot):
          rv = cr[pl.ds(ci * LANES, LANES)]
          bv = cb[pl.ds(ci * LANES, LANES)]
          live = lane_iota < (n_cand - ci * LANES)
          m = jnp.logical_and((rv >= match_lo) & (rv < match_hi), live)
          cnt = _lane0(plsc.all_reduce_population_count(m))

          def process(nslot):
            nslot = lax.cond(nslot + cnt > LANES, flush,
                             lambda s: s, nslot)
            mi = m.astype(jnp.int32)
            slot_vec = nslot + plsc.cumsum(mi) - mi
            col = rv - col_base
            plsc.store_scatter(stageb, [slot_vec], bv, mask=m)
            for f in range(dim):
              fv = jnp.full((LANES,), f, jnp.int32)
              v = plsc.load_gather(bufs, [par_v, fv, col], mask=m)
              plsc.store_scatter(stage, [slot_vec, fv], v, mask=m)
            return nslot + cnt

          return lax.cond(cnt > 0, process, lambda s: s, nslot)

        n_iter = lax.div(n_cand + jnp.int32(LANES - 1), jnp.int32(LANES))
        return lax.fori_loop(0, n_iter, cand_iter, state)

      def compress_segment(lo, hi, start):
        def body(i, carry):
          cnt, resume = carry
          base = i * LANES
          rv = idxv[pl.ds(base, LANES)]
          m = (rv >= lo) & (rv < hi)
          m = jnp.logical_and(m, base >= start)
          npos = _lane0(plsc.all_reduce_population_count(m))

          def with_hits(carry):
            cnt, resume = carry
            ok = jnp.logical_and(cnt + npos <= CAND_CAP,
                                 resume == batch)
            mm = jnp.logical_and(m, ok)
            plsc.store_compressed(cr.at[pl.ds(cnt, LANES)], rv, mask=mm)
            plsc.store_compressed(
                cb.at[pl.ds(cnt, LANES)], lane_iota + base, mask=mm)
            cnt = cnt + npos * ok.astype(jnp.int32)
            resume = jnp.minimum(
                resume, lax.select(ok, jnp.int32(batch), base))
            return cnt, resume

          return lax.cond(npos > 0, with_hits, lambda c: c, carry)

        cnt, resume = lax.fori_loop(
            0, batch // LANES, body, (jnp.int32(0), jnp.int32(batch)))
        return cnt, resume

      state = jnp.int32(0)
      chunk0 = wid * per_tec
      n_seg = len(seg_sizes)

      def seg_body(s, state):
        seg_len = jnp.minimum(
            jnp.int32(SEG_CHUNKS), jnp.int32(per_tec) - s * SEG_CHUNKS)
        lo_chunk = chunk0 + s * SEG_CHUNKS
        lo = lo_chunk * ROWS_PER_CHUNK
        hi = lo + seg_len * ROWS_PER_CHUNK

        def not_done(carry):
          return carry[0] < batch

        def one_pass(carry):
          start, state = carry
          n_cand, resume = compress_segment(lo, hi, start)
          fire(lo_chunk, 0)

          def jbody(j, state):
            par = lax.rem(j, 2)
            lax.cond(j + 1 < seg_len,
                     lambda _: (fire(lo_chunk + j + 1, 1 - par), 0)[1],
                     lambda _: 0, 0)
            wait_chunk(par)
            clo = (lo_chunk + j) * ROWS_PER_CHUNK
            return extract_matches(clo, clo + ROWS_PER_CHUNK, clo,
                                   par, n_cand, state)

          state = lax.fori_loop(0, seg_len, jbody, state)
          return resume, state

        return lax.while_loop(not_done, one_pass, (jnp.int32(0), state))[1]

      state = lax.fori_loop(0, n_seg, seg_body, state)

      if n_extra:
        def leftover_fn(state):
          is_tail = wid == NUM_WORKERS - 1
          full_off = (n_chunks_full - 1) * ROWS_PER_CHUNK
          lo = lax.select(is_tail, jnp.int32(n_chunks_full * ROWS_PER_CHUNK),
                          jnp.int32(full_off))
          hi = lax.select(is_tail, jnp.int32(n_rows),
                          jnp.int32(n_chunks_full * ROWS_PER_CHUNK))
          col_base = lax.select(is_tail, jnp.int32(n_rows - ROWS_PER_CHUNK),
                                jnp.int32(full_off))

          def not_done(carry):
            return carry[0] < batch

          def one_pass(carry):
            start, state = carry
            n_cand, resume = compress_segment(lo, hi, start)

            def dma_full(_):
              pltpu.async_copy(
                  tab_hbm.at[:, pl.ds(full_off, ROWS_PER_CHUNK)],
                  bufs.at[0], sems[0]).wait()
              return 0

            def dma_tail(_):
              pltpu.async_copy(tail_hbm, bufs.at[0], sems[0]).wait()
              return 0

            lax.cond(is_tail, dma_tail, dma_full, 0)
            state = extract_matches(lo, hi, col_base, jnp.int32(0),
                                    n_cand, state)
            return resume, state

          return lax.while_loop(not_done, one_pass,
                                (jnp.int32(0), state))[1]

        state = lax.cond(wid >= NUM_WORKERS - n_extra, leftover_fn,
                         lambda s: s, state)

      lax.cond(state > 0, flush, lambda s: s, state)

    run_table(ut_hbm, tu_hbm, uidx_hbm, eu_hbm)
    run_table(it_hbm, ti_hbm, iidx_hbm, ei_hbm)

  return k


def _make_compute(batch, dim):
  b_per_w = batch // NUM_WORKERS
  half = b_per_w // 2
  mesh = plsc.VectorSubcoreMesh(core_axis_name="c", subcore_axis_name="s")

  @functools.partial(
      pl.kernel,
      out_type=jax.ShapeDtypeStruct((batch,), jnp.float32),
      mesh=mesh,
      compiler_params=pltpu.CompilerParams(
          needs_layout_passes=False, use_tc_tiling_on_sc=True),
      scratch_types=[
          pltpu.VMEM((half, 2 * dim), jnp.float32),
          pltpu.VMEM((half, 2 * dim), jnp.float32),
          pltpu.VMEM((b_per_w,), jnp.float32),
          pltpu.SemaphoreType.DMA,
          pltpu.SemaphoreType.DMA,
      ],
  )
  def k(eu_hbm, ei_hbm, out_hbm, ubuf, ibuf, outv, su, si):
    wid = lax.axis_index("s") * 2 + lax.axis_index("c")
    base = wid * b_per_w
    r_iota = lax.iota(jnp.int32, LANES)

    for h in range(2):
      off = base + h * half
      du = pltpu.async_copy(eu_hbm.at[pl.ds(off, half)], ubuf, su)
      di = pltpu.async_copy(ei_hbm.at[pl.ds(off, half)], ibuf, si)
      du.wait()
      di.wait()

      def block_body(bi, _, h=h):
        rows = r_iota + bi * LANES

        def col_body(j, acc):
          jv = jnp.full((LANES,), 0, jnp.int32) + j
          uvec = plsc.load_gather(ubuf, [rows, jv])
          ivec = plsc.load_gather(ibuf, [rows, jv])
          return acc + uvec * ivec

        acc = lax.fori_loop(0, dim, col_body,
                            jnp.zeros((LANES,), jnp.float32))
        outv[pl.ds(h * half + bi * LANES, LANES)] = acc
        return 0

      lax.fori_loop(0, half // LANES, block_body, 0)

    pltpu.sync_copy(outv, out_hbm.at[pl.ds(base, b_per_w)])

  return k


def kernel(user, item, user_emb, item_emb):
  batch = user.shape[0]
  n_rows, dim = user_emb.shape
  ut = jnp.swapaxes(user_emb, 0, 1)
  it = jnp.swapaxes(item_emb, 0, 1)
  u = user.astype(jnp.int32)
  i = item.astype(jnp.int32)
  tail = n_rows % ROWS_PER_CHUNK
  def tail_stage(tab):
    t = lax.slice(tab, (n_rows - tail, 0), (n_rows, dim))
    t = jnp.swapaxes(t, 0, 1)
    z = jnp.zeros((dim, ROWS_PER_CHUNK), jnp.float32)
    return lax.dynamic_update_slice(z, t, (0, ROWS_PER_CHUNK - tail))
  tu = tail_stage(user_emb)
  ti = tail_stage(item_emb)
  ka = _make_extract(n_rows, dim, batch)
  eu, ei = ka(ut, it, tu, ti, u, i)
  kb = _make_compute(batch, dim)
  return kb(eu, ei)


# speedup vs baseline: 2.2583x; 1.0817x over previous
import functools

import jax
import jax.numpy as jnp
from jax import lax
from jax.experimental import pallas as pl
from jax.experimental.pallas import tpu as pltpu
from jax.experimental.pallas import tpu_sc as plsc

NUM_WORKERS = 32
LANES = 16
ROWS_PER_CHUNK = 512
SEG_CHUNKS = 64
CAND_CAP = 1024

_GATHER_DNUMS = lax.GatherDimensionNumbers(
    offset_dims=(), collapsed_slice_dims=(0,), start_index_map=(0,))


def _splat_lane(vec, lane_splat):
  return lax.gather(
      vec, lane_splat[:, None], _GATHER_DNUMS, (1,),
      mode=lax.GatherScatterMode.PROMISE_IN_BOUNDS)


def _lane0(x):
  return lax.squeeze(lax.slice(x, (0,), (1,)), (0,))


def _make_extract(n_rows, dim, batch):
  n_chunks_full = n_rows // ROWS_PER_CHUNK
  tail = n_rows - n_chunks_full * ROWS_PER_CHUNK
  per_tec = n_chunks_full // NUM_WORKERS
  leftover = n_chunks_full - per_tec * NUM_WORKERS
  n_extra = leftover + (1 if tail else 0)
  assert n_extra <= 2
  mesh = plsc.VectorSubcoreMesh(core_axis_name="c", subcore_axis_name="s")

  seg_sizes = []
  left = per_tec
  while left > 0:
    seg_sizes.append(min(SEG_CHUNKS, left))
    left -= seg_sizes[-1]

  @functools.partial(
      pl.kernel,
      out_type=(jax.ShapeDtypeStruct((batch, 2 * dim), jnp.float32),
                jax.ShapeDtypeStruct((batch, 2 * dim), jnp.float32)),
      mesh=mesh,
      compiler_params=pltpu.CompilerParams(
          needs_layout_passes=False, use_tc_tiling_on_sc=True),
      scratch_types=[
          pltpu.VMEM((batch,), jnp.int32),
          pltpu.VMEM((CAND_CAP + LANES,), jnp.int32),
          pltpu.VMEM((CAND_CAP + LANES,), jnp.int32),
          pltpu.VMEM((2, dim, ROWS_PER_CHUNK), jnp.float32),
          pltpu.VMEM((LANES, 2 * dim), jnp.float32),
          pltpu.VMEM((LANES,), jnp.int32),
          pltpu.SemaphoreType.DMA,
          pltpu.SemaphoreType.DMA,
          pltpu.SemaphoreType.DMA,
      ],
  )
  def k(ut_hbm, it_hbm, tu_hbm, ti_hbm, uidx_hbm, iidx_hbm, eu_hbm, ei_hbm,
        idxv, cr, cb, bufs, stage, stageb, s0, s1, ssc):
    wid = lax.axis_index("s") * 2 + lax.axis_index("c")
    sems = (s0, s1)
    lane_iota = lax.iota(jnp.int32, LANES)
    fdim = dim // LANES

    def run_table(tab_hbm, tail_hbm, idx_hbm, ext_hbm):
      pltpu.sync_copy(idx_hbm, idxv)

      def fire(chunk_id, par):
        off = pl.multiple_of(chunk_id * ROWS_PER_CHUNK, ROWS_PER_CHUNK)

        def go(b):
          def f(_):
            pltpu.async_copy(
                tab_hbm.at[:, pl.ds(off, ROWS_PER_CHUNK)], bufs.at[b],
                sems[b])
            return 0
          return f
        lax.cond(par == 0, go(0), go(1), 0)

      def wait_chunk(par):
        def go(b):
          def f(_):
            pltpu.make_async_copy(
                tab_hbm.at[:, pl.ds(0, ROWS_PER_CHUNK)], bufs.at[b],
                sems[b]).wait()
            return 0
          return f
        lax.cond(par == 0, go(0), go(1), 0)

      def flush(nslot):
        sb = stageb[...]
        stageb[...] = jnp.where(lane_iota < nslot, sb, -1)
        pltpu.async_copy(
            stage, ext_hbm.at[plsc.Indices(stageb, ignored_value=-1)],
            ssc).wait()
        return jnp.int32(0)

      def extract_matches(match_lo, match_hi, col_base, buf_par, n_cand,
                          state):
        par_v = jnp.full((LANES,), 0, jnp.int32) + buf_par

        def cand_iter(ci, nslot):
          rv = cr[pl.ds(ci * LANES, LANES)]
          bv = cb[pl.ds(ci * LANES, LANES)]
          live = lane_iota < (n_cand - ci * LANES)
          m = jnp.logical_and((rv >= match_lo) & (rv < match_hi), live)
          cnt = _lane0(plsc.all_reduce_population_count(m))

          def process(nslot):
            nslot = lax.cond(nslot + cnt > LANES, flush,
                             lambda s: s, nslot)
            mi = m.astype(jnp.int32)
            slot_vec = nslot + plsc.cumsum(mi) - mi
            col = rv - col_base
            plsc.store_scatter(stageb, [slot_vec], bv, mask=m)
            for f in range(dim):
              fv = jnp.full((LANES,), f, jnp.int32)
              v = plsc.load_gather(bufs, [par_v, fv, col], mask=m)
              plsc.store_scatter(stage, [slot_vec, fv], v, mask=m)
            return nslot + cnt

          return lax.cond(cnt > 0, process, lambda s: s, nslot)

        n_iter = lax.div(n_cand + jnp.int32(LANES - 1), jnp.int32(LANES))
        return lax.fori_loop(0, n_iter, cand_iter, state)

      def compress_segment(lo, hi, start):
        def body(i, carry):
          cnt, resume = carry
          base = i * LANES
          rv = idxv[pl.ds(base, LANES)]
          m = (rv >= lo) & (rv < hi)
          m = jnp.logical_and(m, base >= start)
          npos = _lane0(plsc.all_reduce_population_count(m))

          def with_hits(carry):
            cnt, resume = carry
            ok = jnp.logical_and(cnt + npos <= CAND_CAP,
                                 resume == batch)
            mm = jnp.logical_and(m, ok)
            plsc.store_compressed(cr.at[pl.ds(cnt, LANES)], rv, mask=mm)
            plsc.store_compressed(
                cb.at[pl.ds(cnt, LANES)], lane_iota + base, mask=mm)
            cnt = cnt + npos * ok.astype(jnp.int32)
            resume = jnp.minimum(
                resume, lax.select(ok, jnp.int32(batch), base))
            return cnt, resume

          return lax.cond(npos > 0, with_hits, lambda c: c, carry)

        cnt, resume = lax.fori_loop(
            0, batch // LANES, body, (jnp.int32(0), jnp.int32(batch)))
        return cnt, resume

      state = jnp.int32(0)
      chunk0 = wid * per_tec
      n_seg = len(seg_sizes)

      def seg_body(s, state):
        seg_len = jnp.minimum(
            jnp.int32(SEG_CHUNKS), jnp.int32(per_tec) - s * SEG_CHUNKS)
        lo_chunk = chunk0 + s * SEG_CHUNKS
        lo = lo_chunk * ROWS_PER_CHUNK
        hi = lo + seg_len * ROWS_PER_CHUNK

        def not_done(carry):
          return carry[0] < batch

        def one_pass(carry):
          start, state = carry
          n_cand, resume = compress_segment(lo, hi, start)
          fire(lo_chunk, 0)

          def jbody(j, state):
            par = lax.rem(j, 2)
            lax.cond(j + 1 < seg_len,
                     lambda _: (fire(lo_chunk + j + 1, 1 - par), 0)[1],
                     lambda _: 0, 0)
            wait_chunk(par)
            clo = (lo_chunk + j) * ROWS_PER_CHUNK
            return extract_matches(clo, clo + ROWS_PER_CHUNK, clo,
                                   par, n_cand, state)

          state = lax.fori_loop(0, seg_len, jbody, state)
          return resume, state

        return lax.while_loop(not_done, one_pass, (jnp.int32(0), state))[1]

      state = lax.fori_loop(0, n_seg, seg_body, state)

      if n_extra:
        def leftover_fn(state):
          is_tail = wid == NUM_WORKERS - 1
          full_off = (n_chunks_full - 1) * ROWS_PER_CHUNK
          lo = lax.select(is_tail, jnp.int32(n_chunks_full * ROWS_PER_CHUNK),
                          jnp.int32(full_off))
          hi = lax.select(is_tail, jnp.int32(n_rows),
                          jnp.int32(n_chunks_full * ROWS_PER_CHUNK))
          col_base = lax.select(is_tail, jnp.int32(n_rows - ROWS_PER_CHUNK),
                                jnp.int32(full_off))

          def not_done(carry):
            return carry[0] < batch

          def one_pass(carry):
            start, state = carry
            n_cand, resume = compress_segment(lo, hi, start)

            def dma_full(_):
              pltpu.async_copy(
                  tab_hbm.at[:, pl.ds(full_off, ROWS_PER_CHUNK)],
                  bufs.at[0], sems[0]).wait()
              return 0

            def dma_tail(_):
              pltpu.async_copy(tail_hbm, bufs.at[0], sems[0]).wait()
              return 0

            lax.cond(is_tail, dma_tail, dma_full, 0)
            state = extract_matches(lo, hi, col_base, jnp.int32(0),
                                    n_cand, state)
            return resume, state

          return lax.while_loop(not_done, one_pass,
                                (jnp.int32(0), state))[1]

        state = lax.cond(wid >= NUM_WORKERS - n_extra, leftover_fn,
                         lambda s: s, state)

      lax.cond(state > 0, flush, lambda s: s, state)

    run_table(ut_hbm, tu_hbm, uidx_hbm, eu_hbm)
    run_table(it_hbm, ti_hbm, iidx_hbm, ei_hbm)

  return k


def _make_compute(batch, dim):
  b_per_w = batch // NUM_WORKERS
  half = b_per_w // 2
  mesh = plsc.VectorSubcoreMesh(core_axis_name="c", subcore_axis_name="s")

  @functools.partial(
      pl.kernel,
      out_type=jax.ShapeDtypeStruct((batch,), jnp.float32),
      mesh=mesh,
      compiler_params=pltpu.CompilerParams(
          needs_layout_passes=False, use_tc_tiling_on_sc=True),
      scratch_types=[
          pltpu.VMEM((half, 2 * dim), jnp.float32),
          pltpu.VMEM((half, 2 * dim), jnp.float32),
          pltpu.VMEM((b_per_w,), jnp.float32),
          pltpu.SemaphoreType.DMA,
          pltpu.SemaphoreType.DMA,
      ],
  )
  def k(eu_hbm, ei_hbm, out_hbm, ubuf, ibuf, outv, su, si):
    wid = lax.axis_index("s") * 2 + lax.axis_index("c")
    base = wid * b_per_w
    r_iota = lax.iota(jnp.int32, LANES)

    for h in range(2):
      off = base + h * half
      du = pltpu.async_copy(eu_hbm.at[pl.ds(off, half)], ubuf, su)
      di = pltpu.async_copy(ei_hbm.at[pl.ds(off, half)], ibuf, si)
      du.wait()
      di.wait()

      def block_body(bi, _, h=h):
        rows = r_iota + bi * LANES

        def col_body(j, acc):
          jv = jnp.full((LANES,), 0, jnp.int32) + j
          uvec = plsc.load_gather(ubuf, [rows, jv])
          ivec = plsc.load_gather(ibuf, [rows, jv])
          return acc + uvec * ivec

        acc = lax.fori_loop(0, dim, col_body,
                            jnp.zeros((LANES,), jnp.float32))
        outv[pl.ds(h * half + bi * LANES, LANES)] = acc
        return 0

      lax.fori_loop(0, half // LANES, block_body, 0)

    pltpu.sync_copy(outv, out_hbm.at[pl.ds(base, b_per_w)])

  return k


def kernel(user, item, user_emb, item_emb):
  batch = user.shape[0]
  n_rows, dim = user_emb.shape
  ut = jnp.swapaxes(user_emb, 0, 1)
  it = jnp.swapaxes(item_emb, 0, 1)
  u = user.astype(jnp.int32)
  i = item.astype(jnp.int32)
  tail = n_rows % ROWS_PER_CHUNK
  def tail_stage(tab):
    t = lax.slice(tab, (n_rows - tail, 0), (n_rows, dim))
    t = jnp.swapaxes(t, 0, 1)
    z = jnp.zeros((dim, ROWS_PER_CHUNK), jnp.float32)
    return lax.dynamic_update_slice(z, t, (0, ROWS_PER_CHUNK - tail))
  tu = tail_stage(user_emb)
  ti = tail_stage(item_emb)
  ka = _make_extract(n_rows, dim, batch)
  eu, ei = ka(ut, it, tu, ti, u, i)
  kb = _make_compute(batch, dim)
  return kb(eu, ei)


# speedup vs baseline: 3.0036x; 1.3301x over previous
import functools

import jax
import jax.numpy as jnp
from jax import lax
from jax.experimental import pallas as pl
from jax.experimental.pallas import tpu as pltpu
from jax.experimental.pallas import tpu_sc as plsc

NUM_WORKERS = 32
LANES = 16
ROWS_PER_CHUNK = 512
SEG_CHUNKS = 64
CAND_CAP = 1024

_GATHER_DNUMS = lax.GatherDimensionNumbers(
    offset_dims=(), collapsed_slice_dims=(0,), start_index_map=(0,))


def _splat_lane(vec, lane_splat):
  return lax.gather(
      vec, lane_splat[:, None], _GATHER_DNUMS, (1,),
      mode=lax.GatherScatterMode.PROMISE_IN_BOUNDS)


def _gather16(vec, idx):
  return lax.gather(
      vec, idx[:, None], _GATHER_DNUMS, (1,),
      mode=lax.GatherScatterMode.PROMISE_IN_BOUNDS)


def _lane0(x):
  return lax.squeeze(lax.slice(x, (0,), (1,)), (0,))


def _make_extract(n_rows, dim, batch):
  n_chunks_full = n_rows // ROWS_PER_CHUNK
  tail = n_rows - n_chunks_full * ROWS_PER_CHUNK
  per_tec = n_chunks_full // NUM_WORKERS
  leftover = n_chunks_full - per_tec * NUM_WORKERS
  n_extra = leftover + (1 if tail else 0)
  assert n_extra <= 2
  mesh = plsc.VectorSubcoreMesh(core_axis_name="c", subcore_axis_name="s")

  seg_sizes = []
  left = per_tec
  while left > 0:
    seg_sizes.append(min(SEG_CHUNKS, left))
    left -= seg_sizes[-1]

  @functools.partial(
      pl.kernel,
      out_type=(jax.ShapeDtypeStruct((batch, 2 * dim), jnp.float32),
                jax.ShapeDtypeStruct((batch, 2 * dim), jnp.float32)),
      mesh=mesh,
      compiler_params=pltpu.CompilerParams(
          needs_layout_passes=False, use_tc_tiling_on_sc=True),
      scratch_types=[
          pltpu.VMEM((batch,), jnp.int32),
          pltpu.VMEM((CAND_CAP + LANES,), jnp.int32),
          pltpu.VMEM((CAND_CAP + LANES,), jnp.int32),
          pltpu.VMEM((2, dim, ROWS_PER_CHUNK), jnp.float32),
          pltpu.VMEM((LANES, 2 * dim), jnp.float32),
          pltpu.VMEM((LANES,), jnp.int32),
          pltpu.VMEM((64 * 32,), jnp.int32),
          pltpu.VMEM((64 * 32,), jnp.int32),
          pltpu.VMEM((64,), jnp.int32),
          pltpu.SemaphoreType.DMA,
          pltpu.SemaphoreType.DMA,
          pltpu.SemaphoreType.DMA,
      ],
  )
  def k(ut_hbm, it_hbm, tu_hbm, ti_hbm, uidx_hbm, iidx_hbm, eu_hbm, ei_hbm,
        idxv, cr, cb, bufs, stage, stageb, rbin, bbin, bincnt, s0, s1, ssc):
    wid = lax.axis_index("s") * 2 + lax.axis_index("c")
    sems = (s0, s1)
    lane_iota = lax.iota(jnp.int32, LANES)
    fdim = dim // LANES

    def run_table(tab_hbm, tail_hbm, idx_hbm, ext_hbm):
      pltpu.sync_copy(idx_hbm, idxv)

      def fire(chunk_id, par):
        off = pl.multiple_of(chunk_id * ROWS_PER_CHUNK, ROWS_PER_CHUNK)

        def go(b):
          def f(_):
            pltpu.async_copy(
                tab_hbm.at[:, pl.ds(off, ROWS_PER_CHUNK)], bufs.at[b],
                sems[b])
            return 0
          return f
        lax.cond(par == 0, go(0), go(1), 0)

      def wait_chunk(par):
        def go(b):
          def f(_):
            pltpu.make_async_copy(
                tab_hbm.at[:, pl.ds(0, ROWS_PER_CHUNK)], bufs.at[b],
                sems[b]).wait()
            return 0
          return f
        lax.cond(par == 0, go(0), go(1), 0)

      def flush(nslot):
        sb = stageb[...]
        stageb[...] = jnp.where(lane_iota < nslot, sb, -1)
        pltpu.async_copy(
            stage, ext_hbm.at[plsc.Indices(stageb, ignored_value=-1)],
            ssc).wait()
        return jnp.int32(0)

      def process_vreg(rv, bv, m, buf_par, col_base, nslot):
        par_v = jnp.full((LANES,), 0, jnp.int32) + buf_par
        cnt = _lane0(plsc.all_reduce_population_count(m))

        def process(nslot):
          nslot = lax.cond(nslot + cnt > LANES, flush, lambda s: s, nslot)
          mi = m.astype(jnp.int32)
          slot_vec = nslot + plsc.cumsum(mi) - mi
          col = rv - col_base
          plsc.store_scatter(stageb, [slot_vec], bv, mask=m)
          for f in range(dim):
            fv = jnp.full((LANES,), f, jnp.int32)
            v = plsc.load_gather(bufs, [par_v, fv, col], mask=m)
            plsc.store_scatter(stage, [slot_vec, fv], v, mask=m)
          return nslot + cnt

        return lax.cond(cnt > 0, process, lambda s: s, nslot)

      def extract_matches(match_lo, match_hi, col_base, buf_par, n_cand,
                          state):
        def cand_iter(ci, nslot):
          rv = cr[pl.ds(ci * LANES, LANES)]
          bv = cb[pl.ds(ci * LANES, LANES)]
          live = lane_iota < (n_cand - ci * LANES)
          m = jnp.logical_and((rv >= match_lo) & (rv < match_hi), live)
          return process_vreg(rv, bv, m, buf_par, col_base, nslot)

        n_iter = lax.div(n_cand + jnp.int32(LANES - 1), jnp.int32(LANES))
        return lax.fori_loop(0, n_iter, cand_iter, state)

      def build_bins(lo, n_cand):
        for z in range(4):
          bincnt[pl.ds(z * LANES, LANES)] = jnp.zeros((LANES,), jnp.int32)

        def bin_iter(ci, _):
          rv = cr[pl.ds(ci * LANES, LANES)]
          bv = cb[pl.ds(ci * LANES, LANES)]
          live = lane_iota < (n_cand - ci * LANES)
          li = live.astype(jnp.int32)
          c = jnp.where(live, (rv - lo) // ROWS_PER_CHUNK, 63)
          rank = jnp.zeros((LANES,), jnp.int32)
          ups = jnp.zeros((LANES,), jnp.int32)
          for kk in range(1, LANES):
            idn = jnp.where(lane_iota >= kk, lane_iota - kk, 0)
            eqd = (_gather16(c, idn) == c) & (lane_iota >= kk)
            eqd = eqd & (_gather16(li, idn) > 0)
            rank = rank + eqd.astype(jnp.int32)
            idu = jnp.where(lane_iota < LANES - kk, lane_iota + kk, 0)
            equ = (_gather16(c, idu) == c) & (lane_iota < LANES - kk)
            equ = equ & (_gather16(li, idu) > 0)
            ups = ups + equ.astype(jnp.int32)
          base = plsc.load_gather(bincnt, [c])
          plsc.store_scatter(bincnt, [c], base + rank + ups + 1, mask=live)
          pos = c * 32 + base + rank
          fit = jnp.logical_and(live, (base + rank) < 32)
          plsc.store_scatter(rbin, [pos], rv, mask=fit)
          plsc.store_scatter(bbin, [pos], bv, mask=fit)
          return 0

        n_iter = lax.div(n_cand + jnp.int32(LANES - 1), jnp.int32(LANES))
        lax.fori_loop(0, n_iter, bin_iter, 0)

      def compress_segment(lo, hi, start):
        def body(i, carry):
          cnt, resume = carry
          base = i * LANES
          rv = idxv[pl.ds(base, LANES)]
          m = (rv >= lo) & (rv < hi)
          m = jnp.logical_and(m, base >= start)
          npos = _lane0(plsc.all_reduce_population_count(m))

          def with_hits(carry):
            cnt, resume = carry
            ok = jnp.logical_and(cnt + npos <= CAND_CAP,
                                 resume == batch)
            mm = jnp.logical_and(m, ok)
            plsc.store_compressed(cr.at[pl.ds(cnt, LANES)], rv, mask=mm)
            plsc.store_compressed(
                cb.at[pl.ds(cnt, LANES)], lane_iota + base, mask=mm)
            cnt = cnt + npos * ok.astype(jnp.int32)
            resume = jnp.minimum(
                resume, lax.select(ok, jnp.int32(batch), base))
            return cnt, resume

          return lax.cond(npos > 0, with_hits, lambda c: c, carry)

        cnt, resume = lax.fori_loop(
            0, batch // LANES, body, (jnp.int32(0), jnp.int32(batch)))
        return cnt, resume

      state = jnp.int32(0)
      chunk0 = wid * per_tec
      n_seg = len(seg_sizes)

      def seg_body(s, state):
        seg_len = jnp.minimum(
            jnp.int32(SEG_CHUNKS), jnp.int32(per_tec) - s * SEG_CHUNKS)
        lo_chunk = chunk0 + s * SEG_CHUNKS
        lo = lo_chunk * ROWS_PER_CHUNK
        hi = lo + seg_len * ROWS_PER_CHUNK

        def not_done(carry):
          return carry[0] < batch

        def one_pass(carry):
          start, state = carry
          n_cand, resume = compress_segment(lo, hi, start)
          fire(lo_chunk, 0)

          build_bins(lo, n_cand)

          def jbody(j, state):
            par = lax.rem(j, 2)
            lax.cond(j + 1 < seg_len,
                     lambda _: (fire(lo_chunk + j + 1, 1 - par), 0)[1],
                     lambda _: 0, 0)
            wait_chunk(par)
            clo = (lo_chunk + j) * ROWS_PER_CHUNK
            nc = _lane0(plsc.load_gather(
                bincnt, [jnp.full((LANES,), 0, jnp.int32) + j]))

            def bucket_path(nslot):
              for v in range(2):
                rv = rbin[pl.ds(j * 32 + v * LANES, LANES)]
                bv = bbin[pl.ds(j * 32 + v * LANES, LANES)]
                m = lane_iota < (nc - v * LANES)
                nslot = process_vreg(rv, bv, m, par, clo, nslot)
              return nslot

            def fallback(nslot):
              return extract_matches(clo, clo + ROWS_PER_CHUNK, clo,
                                     par, n_cand, nslot)

            return lax.cond(nc <= 32, bucket_path, fallback, state)

          state = lax.fori_loop(0, seg_len, jbody, state)
          return resume, state

        return lax.while_loop(not_done, one_pass, (jnp.int32(0), state))[1]

      state = lax.fori_loop(0, n_seg, seg_body, state)

      if n_extra:
        def leftover_fn(state):
          is_tail = wid == NUM_WORKERS - 1
          full_off = (n_chunks_full - 1) * ROWS_PER_CHUNK
          lo = lax.select(is_tail, jnp.int32(n_chunks_full * ROWS_PER_CHUNK),
                          jnp.int32(full_off))
          hi = lax.select(is_tail, jnp.int32(n_rows),
                          jnp.int32(n_chunks_full * ROWS_PER_CHUNK))
          col_base = lax.select(is_tail, jnp.int32(n_rows - ROWS_PER_CHUNK),
                                jnp.int32(full_off))

          def not_done(carry):
            return carry[0] < batch

          def one_pass(carry):
            start, state = carry
            n_cand, resume = compress_segment(lo, hi, start)

            def dma_full(_):
              pltpu.async_copy(
                  tab_hbm.at[:, pl.ds(full_off, ROWS_PER_CHUNK)],
                  bufs.at[0], sems[0]).wait()
              return 0

            def dma_tail(_):
              pltpu.async_copy(tail_hbm, bufs.at[0], sems[0]).wait()
              return 0

            lax.cond(is_tail, dma_tail, dma_full, 0)
            state = extract_matches(lo, hi, col_base, jnp.int32(0),
                                    n_cand, state)
            return resume, state

          return lax.while_loop(not_done, one_pass,
                                (jnp.int32(0), state))[1]

        state = lax.cond(wid >= NUM_WORKERS - n_extra, leftover_fn,
                         lambda s: s, state)

      lax.cond(state > 0, flush, lambda s: s, state)

    run_table(ut_hbm, tu_hbm, uidx_hbm, eu_hbm)
    run_table(it_hbm, ti_hbm, iidx_hbm, ei_hbm)

  return k


def _make_compute(batch, dim):
  b_per_w = batch // NUM_WORKERS
  half = b_per_w // 2
  mesh = plsc.VectorSubcoreMesh(core_axis_name="c", subcore_axis_name="s")

  @functools.partial(
      pl.kernel,
      out_type=jax.ShapeDtypeStruct((batch,), jnp.float32),
      mesh=mesh,
      compiler_params=pltpu.CompilerParams(
          needs_layout_passes=False, use_tc_tiling_on_sc=True),
      scratch_types=[
          pltpu.VMEM((half, 2 * dim), jnp.float32),
          pltpu.VMEM((half, 2 * dim), jnp.float32),
          pltpu.VMEM((b_per_w,), jnp.float32),
          pltpu.SemaphoreType.DMA,
          pltpu.SemaphoreType.DMA,
      ],
  )
  def k(eu_hbm, ei_hbm, out_hbm, ubuf, ibuf, outv, su, si):
    wid = lax.axis_index("s") * 2 + lax.axis_index("c")
    base = wid * b_per_w
    r_iota = lax.iota(jnp.int32, LANES)

    for h in range(2):
      off = base + h * half
      du = pltpu.async_copy(eu_hbm.at[pl.ds(off, half)], ubuf, su)
      di = pltpu.async_copy(ei_hbm.at[pl.ds(off, half)], ibuf, si)
      du.wait()
      di.wait()

      def block_body(bi, _, h=h):
        rows = r_iota + bi * LANES

        def col_body(j, acc):
          jv = jnp.full((LANES,), 0, jnp.int32) + j
          uvec = plsc.load_gather(ubuf, [rows, jv])
          ivec = plsc.load_gather(ibuf, [rows, jv])
          return acc + uvec * ivec

        acc = lax.fori_loop(0, dim, col_body,
                            jnp.zeros((LANES,), jnp.float32))
        outv[pl.ds(h * half + bi * LANES, LANES)] = acc
        return 0

      lax.fori_loop(0, half // LANES, block_body, 0)

    pltpu.sync_copy(outv, out_hbm.at[pl.ds(base, b_per_w)])

  return k


def kernel(user, item, user_emb, item_emb):
  batch = user.shape[0]
  n_rows, dim = user_emb.shape
  ut = jnp.swapaxes(user_emb, 0, 1)
  it = jnp.swapaxes(item_emb, 0, 1)
  u = user.astype(jnp.int32)
  i = item.astype(jnp.int32)
  tail = n_rows % ROWS_PER_CHUNK
  def tail_stage(tab):
    t = lax.slice(tab, (n_rows - tail, 0), (n_rows, dim))
    t = jnp.swapaxes(t, 0, 1)
    z = jnp.zeros((dim, ROWS_PER_CHUNK), jnp.float32)
    return lax.dynamic_update_slice(z, t, (0, ROWS_PER_CHUNK - tail))
  tu = tail_stage(user_emb)
  ti = tail_stage(item_emb)
  ka = _make_extract(n_rows, dim, batch)
  eu, ei = ka(ut, it, tu, ti, u, i)
  kb = _make_compute(batch, dim)
  return kb(eu, ei)
